# baseline (speedup 1.0000x reference)
import functools

import jax
import jax.numpy as jnp
from jax import lax
from jax.experimental import pallas as pl
from jax.experimental.pallas import tpu as pltpu
from jax.experimental.pallas import tpu_sc as plsc

_B = 2
_C = 32
_N = 4096
_OUTROW = _C * 64
_CQ = 8
_QCOL = _CQ * 64
_NBLOCKS = (_B * _N) // 128
_BLK_PER_W = _NBLOCKS // 32


@functools.partial(
    pl.kernel,
    out_type=jax.ShapeDtypeStruct((_B * _N, _OUTROW), jnp.float32),
    mesh=plsc.VectorSubcoreMesh(core_axis_name="c", subcore_axis_name="s"),
    compiler_params=pltpu.CompilerParams(needs_layout_passes=False),
    scratch_types=[
        pltpu.VMEM((6, 6, _CQ, 128), jnp.float32),
        pltpu.VMEM((128, _QCOL), jnp.float32),
        pltpu.SemaphoreType.DMA,
        pltpu.SemaphoreType.DMA,
    ],
)
def _sc_transpose(x_hbm, out_hbm, in_b, out_b, sem_in, sem_out):
    cid = lax.axis_index("c")
    sid = lax.axis_index("s")
    wid = sid * 2 + cid

    rows16 = lax.iota(jnp.int32, 16)
    zeros16 = jnp.zeros((16,), jnp.float32)

    def zrow(r, carry):
        def zcol(j, inner):
            out_b[r, pl.ds(j * 16, 16)] = zeros16
            return inner

        return lax.fori_loop(0, _QCOL // 16, zcol, carry)

    lax.fori_loop(0, 128, zrow, 0)

    def block_body(k, carry):
        blk = wid * _BLK_PER_W + k
        b = blk // 32
        nblk = blk % 32
        n0 = pl.multiple_of(nblk * 128, 128)
        base = pl.multiple_of(b * _N + nblk * 128, 128)

        for q in range(4):
            c0 = q * _CQ
            pltpu.async_copy(
                x_hbm.at[b, pl.ds(1, 6), pl.ds(1, 6), pl.ds(c0, _CQ), pl.ds(n0, 128)],
                in_b,
                sem_in,
            ).wait()

            def s_body(s, cc):
                s16 = pl.multiple_of(s * 16, 16)
                rows = rows16 + s16
                for c in range(_CQ):
                    for g in range(6):
                        for rj in range(6):
                            col = c * 64 + (g + 1) * 8 + (rj + 1)
                            vec = in_b[g, rj, c, pl.ds(s16, 16)]
                            plsc.store_scatter(out_b, [rows, rows16 * 0 + col], vec)
                return cc

            lax.fori_loop(0, 8, s_body, 0)
            pltpu.sync_copy(
                out_b, out_hbm.at[pl.ds(base, 128), pl.ds(q * _QCOL, _QCOL)]
            )
        return carry

    lax.fori_loop(0, _BLK_PER_W, block_body, 0)


def kernel(x):
    x5 = x.reshape(_B, 8, 8, _C, _N)
    out = _sc_transpose(x5)
    return out.reshape(_B, 16, 16, 16, _C, 8, 8)


# speedup vs baseline: 1.4384x; 1.4384x over previous
import functools

import jax
import jax.numpy as jnp
from jax import lax
from jax.experimental import pallas as pl
from jax.experimental.pallas import tpu as pltpu
from jax.experimental.pallas import tpu_sc as plsc

_C = 32
_N = 4096
_NW = 32
_NPW = _N // _NW
_SUB = 8
_OUTROW = _C * 64


@functools.partial(
    pl.kernel,
    out_type=jax.ShapeDtypeStruct((2 * _N, _OUTROW), jnp.float32),
    mesh=plsc.VectorSubcoreMesh(core_axis_name="c", subcore_axis_name="s"),
    compiler_params=pltpu.CompilerParams(needs_layout_passes=False),
    scratch_types=[
        pltpu.VMEM((_C, _SUB, 128), jnp.float32),
        pltpu.VMEM((_SUB, _OUTROW), jnp.float32),
        pltpu.VMEM((_SUB, _OUTROW), jnp.float32),
        pltpu.SemaphoreType.DMA,
        pltpu.SemaphoreType.DMA,
    ],
)
def _sc_transpose(x_hbm, out_hbm, in_b, ob0, ob1, sem_in, sem_out):
    cid = lax.axis_index("c")
    sid = lax.axis_index("s")
    wid = sid * 2 + cid
    n0w = wid * _NPW

    t16 = lax.iota(jnp.int32, 16)
    jvec = t16 & 7
    mask = (jvec != 0) & (jvec != 7)
    zeros16 = jnp.zeros((16,), jnp.float32)

    def zrow(r, carry):
        def zcol(u, inner):
            ob0[r, pl.ds(u * 16, 16)] = zeros16
            ob1[r, pl.ds(u * 16, 16)] = zeros16
            return inner

        return lax.fori_loop(0, _OUTROW // 16, zcol, carry)

    lax.fori_loop(0, _SUB, zrow, 0)

    def sub_body(s, carry):
        nn = pl.multiple_of(n0w + s * _SUB, _SUB)
        pltpu.async_copy(x_hbm.at[:, pl.ds(nn, _SUB), :], in_b, sem_in).wait()

        def c_body(c, inner):
            cbase = c * 64
            for nl in range(_SUB):
                for k in range(3):
                    off = 8 + 16 * k
                    v0 = in_b[c, nl, pl.ds(off, 16)]
                    ob0[nl, pl.ds(cbase + off, 16)] = jnp.where(mask, v0, zeros16)
                    v1 = in_b[c, nl, pl.ds(64 + off, 16)]
                    ob1[nl, pl.ds(cbase + off, 16)] = jnp.where(mask, v1, zeros16)
            return inner

        lax.fori_loop(0, _C, c_body, 0)

        pltpu.sync_copy(ob0, out_hbm.at[pl.ds(nn, _SUB), :])
        pltpu.sync_copy(ob1, out_hbm.at[pl.ds(_N + nn, _SUB), :])
        return carry

    lax.fori_loop(0, _NPW // _SUB, sub_body, 0)


def kernel(x):
    xv = jnp.transpose(x, (1, 2, 3, 4, 0)).reshape(_C, _N, 128)
    out = _sc_transpose(xv)
    return out.reshape(2, 16, 16, 16, _C, 8, 8)


# speedup vs baseline: 1.5129x; 1.0518x over previous
import functools

import jax
import jax.numpy as jnp
from jax import lax
from jax.experimental import pallas as pl
from jax.experimental.pallas import tpu as pltpu
from jax.experimental.pallas import tpu_sc as plsc

_C = 32
_N = 4096
_NW = 32
_NPW = _N // _NW
_SUB = 8
_NSUB = _NPW // _SUB
_OUTROW = _C * 64


@functools.partial(
    pl.kernel,
    out_type=jax.ShapeDtypeStruct((2 * _N, _OUTROW), jnp.float32),
    mesh=plsc.VectorSubcoreMesh(core_axis_name="c", subcore_axis_name="s"),
    compiler_params=pltpu.CompilerParams(needs_layout_passes=False),
    scratch_types=[
        pltpu.VMEM((2, _C, _SUB, 128), jnp.float32),
        pltpu.VMEM((2, 2, _SUB, _OUTROW), jnp.float32),
        pltpu.SemaphoreType.DMA,
        pltpu.SemaphoreType.DMA,
        pltpu.SemaphoreType.DMA,
        pltpu.SemaphoreType.DMA,
    ],
)
def _sc_transpose(x_hbm, out_hbm, in_b, ob, sem_in0, sem_in1, sem_out0, sem_out1):
    cid = lax.axis_index("c")
    sid = lax.axis_index("s")
    wid = sid * 2 + cid
    n0w = wid * _NPW

    sem_in = (sem_in0, sem_in1)
    sem_out = (sem_out0, sem_out1)

    t16 = lax.iota(jnp.int32, 16)
    jvec = t16 & 7
    mask = (jvec != 0) & (jvec != 7)
    zeros16 = jnp.zeros((16,), jnp.float32)

    def zrow(r, carry):
        def zcol(u, inner):
            ob[0, 0, r, pl.ds(u * 16, 16)] = zeros16
            ob[0, 1, r, pl.ds(u * 16, 16)] = zeros16
            ob[1, 0, r, pl.ds(u * 16, 16)] = zeros16
            ob[1, 1, r, pl.ds(u * 16, 16)] = zeros16
            return inner

        return lax.fori_loop(0, _OUTROW // 16, zcol, carry)

    lax.fori_loop(0, _SUB, zrow, 0)

    def issue_in(s):
        p = s % 2
        nn = pl.multiple_of(n0w + s * _SUB, _SUB)
        return pltpu.async_copy(
            x_hbm.at[:, pl.ds(nn, _SUB), :], in_b.at[p], sem_in[p]
        )

    in_copies = [None, None]
    out_copies = [[None, None], [None, None]]
    in_copies[0] = issue_in(0)

    for s in range(_NSUB):
        p = s % 2
        if s + 1 < _NSUB:
            in_copies[(s + 1) % 2] = issue_in(s + 1)
        in_copies[p].wait()
        if s >= 2:
            out_copies[p][0].wait()
            out_copies[p][1].wait()

        def c_body(c, inner, _p=p):
            cbase = c * 64
            for nl in range(_SUB):
                for k in range(3):
                    off = 8 + 16 * k
                    v0 = in_b[_p, c, nl, pl.ds(off, 16)]
                    ob[_p, 0, nl, pl.ds(cbase + off, 16)] = jnp.where(
                        mask, v0, zeros16
                    )
                    v1 = in_b[_p, c, nl, pl.ds(64 + off, 16)]
                    ob[_p, 1, nl, pl.ds(cbase + off, 16)] = jnp.where(
                        mask, v1, zeros16
                    )
            return inner

        lax.fori_loop(0, _C, c_body, 0)

        nn = pl.multiple_of(n0w + s * _SUB, _SUB)
        out_copies[p][0] = pltpu.async_copy(
            ob.at[p, 0], out_hbm.at[pl.ds(nn, _SUB), :], sem_out[p]
        )
        out_copies[p][1] = pltpu.async_copy(
            ob.at[p, 1], out_hbm.at[pl.ds(_N + nn, _SUB), :], sem_out[p]
        )

    for p in range(2):
        out_copies[p][0].wait()
        out_copies[p][1].wait()


def kernel(x):
    xv = jnp.transpose(x, (1, 2, 3, 4, 0)).reshape(_C, _N, 128)
    out = _sc_transpose(xv)
    return out.reshape(2, 16, 16, 16, _C, 8, 8)
